# baseline (speedup 1.0000x reference)
import functools
import inspect

import jax
import jax.numpy as jnp
from jax import lax
from jax.experimental import pallas as pl
from jax.experimental.pallas import tpu as pltpu
from jax.experimental.pallas import tpu_sc as plsc

N = 10000
D = 128
E = 320000
NC, NS = 2, 16
NW = NC * NS
CHUNK = 88
NBUF = 4
NIDX = 8
CHUNKS_PER_TILE = 120
PER_TILE = CHUNKS_PER_TILE * CHUNK
E_PAD = PER_TILE * NW
N_ACC = 10112
ROWS_PER_SUB = N_ACC // NS

_BM = 1000


def _ln_relu_body(x_ref, g_ref, b_ref, o_ref):
    x = x_ref[...]
    mu = jnp.mean(x, axis=1, keepdims=True)
    var = jnp.mean((x - mu) ** 2, axis=1, keepdims=True)
    h = (x - mu) * lax.rsqrt(var + 1e-5) * g_ref[...] + b_ref[...]
    o_ref[...] = jnp.maximum(h, 0.0)


def _ln_relu(x, g, b):
    return pl.pallas_call(
        _ln_relu_body,
        out_shape=jax.ShapeDtypeStruct((N, D), jnp.float32),
        grid=(N // _BM,),
        in_specs=[
            pl.BlockSpec((_BM, D), lambda i: (i, 0)),
            pl.BlockSpec((1, D), lambda i: (0, 0)),
            pl.BlockSpec((1, D), lambda i: (0, 0)),
        ],
        out_specs=pl.BlockSpec((_BM, D), lambda i: (i, 0)),
    )(x, g, b)


def _make_mesh():
    kw = {}
    params = inspect.signature(plsc.VectorSubcoreMesh).parameters
    if "num_cores" in params:
        kw["num_cores"] = NC
    if "num_subcores" in params:
        kw["num_subcores"] = NS
    return plsc.VectorSubcoreMesh(core_axis_name="c", subcore_axis_name="s", **kw)


def _sc_agg(h, src, dst, zrows, zdeg):
    @functools.partial(
        pl.kernel,
        out_type=[
            jax.ShapeDtypeStruct((NC, N_ACC, D), jnp.float32),
            jax.ShapeDtypeStruct((NC * N_ACC,), jnp.float32),
        ],
        mesh=_make_mesh(),
        scratch_types=(
            [pltpu.VMEM((CHUNK,), jnp.int32)] * NIDX
            + [pltpu.VMEM((CHUNK,), jnp.int32)] * NIDX
            + [pltpu.VMEM((CHUNK, D), jnp.float32)] * NBUF
            + [
                pltpu.VMEM((CHUNK,), jnp.float32),
                pltpu.VMEM_SHARED((N_ACC, D), jnp.float32),
                pltpu.VMEM_SHARED((N_ACC,), jnp.float32),
            ]
            + [pltpu.SemaphoreType.DMA] * (NIDX + 2 * NBUF)
        ),
    )
    def k(h_hbm, src_hbm, dst_hbm, zr_hbm, zd_hbm, out_hbm, deg_hbm, *refs):
        sidx = refs[:NIDX]
        didx = refs[NIDX:2 * NIDX]
        rows = refs[2 * NIDX:2 * NIDX + NBUF]
        ones = refs[2 * NIDX + NBUF]
        acc = refs[2 * NIDX + NBUF + 1]
        accd = refs[2 * NIDX + NBUF + 2]
        sembase = 2 * NIDX + NBUF + 3
        isem = refs[sembase:sembase + NIDX]
        gsem = refs[sembase + NIDX:sembase + NIDX + NBUF]
        ssem = refs[sembase + NIDX + NBUF:sembase + NIDX + 2 * NBUF]
        cid = lax.axis_index("c")
        sid = lax.axis_index("s")
        for i in range(0, CHUNK - 15, 16):
            ones[pl.ds(i, 16)] = jnp.full((16,), 1.0, jnp.float32)
        if CHUNK % 16:
            ones[pl.ds(CHUNK - 16, 16)] = jnp.full((16,), 1.0, jnp.float32)
        zoff = sid * ROWS_PER_SUB
        pltpu.sync_copy(zr_hbm, acc.at[pl.ds(zoff, ROWS_PER_SUB)])

        @pl.when(sid == 0)
        def _():
            pltpu.sync_copy(zd_hbm, accd)

        plsc.subcore_barrier()

        base = cid * (NS * PER_TILE) + sid * PER_TILE

        def idx_load(c, j, sync):
            s_slice = src_hbm.at[pl.ds(base + c * CHUNK, CHUNK)]
            d_slice = dst_hbm.at[pl.ds(base + c * CHUNK, CHUNK)]
            if sync:
                pltpu.sync_copy(s_slice, sidx[j])
                pltpu.sync_copy(d_slice, didx[j])
            else:
                pltpu.async_copy(s_slice, sidx[j], isem[j])
                pltpu.async_copy(d_slice, didx[j], isem[j])

        def idx_wait(j):
            pltpu.make_async_copy(src_hbm.at[pl.ds(0, CHUNK)], sidx[j],
                                  isem[j]).wait()
            pltpu.make_async_copy(dst_hbm.at[pl.ds(0, CHUNK)], didx[j],
                                  isem[j]).wait()

        for j in range(NBUF - 1):
            idx_load(j, j, sync=True)
        for j in range(NBUF - 1, NIDX):
            idx_load(j, j, sync=False)
        for b in range(NBUF - 1):
            pltpu.async_copy(h_hbm.at[sidx[b]], rows[b], gsem[b])

        @pl.loop(0, CHUNKS_PER_TILE, step=NIDX)
        def _(c0):
            for j in range(NIDX):
                c = c0 + j
                b = j % NBUF
                bp = (j - 1) % NBUF
                pltpu.make_async_copy(h_hbm.at[pl.ds(0, CHUNK)], rows[b],
                                      gsem[b]).wait()
                pltpu.async_copy(rows[b], acc.at[didx[j]], ssem[b], add=True)
                pltpu.sync_copy(ones, accd.at[didx[j]], add=True)

                @pl.when(c + NBUF - 1 < CHUNKS_PER_TILE)
                def _():
                    @pl.when(c > 0)
                    def _():
                        pltpu.make_async_copy(rows[bp], acc.at[didx[j]],
                                              ssem[bp]).wait()
                    idx_wait((j + NBUF - 1) % NIDX)
                    pltpu.async_copy(h_hbm.at[sidx[(j + NBUF - 1) % NIDX]],
                                     rows[bp], gsem[bp])

                    @pl.when((c > 0) & (c - 1 + NIDX < CHUNKS_PER_TILE))
                    def _():
                        idx_load(c - 1 + NIDX, (j - 1) % NIDX, sync=False)

        for t in range(NBUF):
            b = (CHUNKS_PER_TILE - 1 - t) % NBUF
            pltpu.make_async_copy(rows[b], acc.at[didx[0]], ssem[b]).wait()

        plsc.subcore_barrier()
        pltpu.sync_copy(
            acc.at[pl.ds(zoff, ROWS_PER_SUB)],
            out_hbm.at[cid, pl.ds(zoff, ROWS_PER_SUB)],
        )
        @pl.when(sid == 0)
        def _():
            pltpu.sync_copy(accd, deg_hbm.at[pl.ds(cid * N_ACC, N_ACC)])

    return k(h, src, dst, zrows, zdeg)


def _combine_body(p_ref, d_ref, h_ref, wl_ref, bl_ref, wr_ref, o_ref):
    p = p_ref[...]
    a = p[0] + p[1]
    d = d_ref[...]
    deg = jnp.maximum(d[0] + d[1], 1.0)
    mean = a / deg
    dn = (((1,), (1,)), ((), ()))
    o_ref[...] = (
        lax.dot_general(mean, wl_ref[...], dn, precision=lax.Precision.HIGHEST,
                        preferred_element_type=jnp.float32)
        + lax.dot_general(h_ref[...], wr_ref[...], dn,
                          precision=lax.Precision.HIGHEST,
                          preferred_element_type=jnp.float32)
        + bl_ref[...]
    )


def _combine(parts, degs, h, W_l, b_l, W_r):
    return pl.pallas_call(
        _combine_body,
        out_shape=jax.ShapeDtypeStruct((N, D), jnp.float32),
        grid=(N // _BM,),
        in_specs=[
            pl.BlockSpec((NC, _BM, D), lambda i: (0, i, 0)),
            pl.BlockSpec((NC, _BM, 1), lambda i: (0, i, 0)),
            pl.BlockSpec((_BM, D), lambda i: (i, 0)),
            pl.BlockSpec((D, D), lambda i: (0, 0)),
            pl.BlockSpec((1, D), lambda i: (0, 0)),
            pl.BlockSpec((D, D), lambda i: (0, 0)),
        ],
        out_specs=pl.BlockSpec((_BM, D), lambda i: (i, 0)),
    )(parts, degs, h, W_l, b_l, W_r)


def kernel(x, edge_index, ln_gamma, ln_beta, W_l, b_l, W_r):
    src = edge_index[0].astype(jnp.int32)
    dst = edge_index[1].astype(jnp.int32)
    pad = E_PAD - E
    src = jnp.concatenate([src, jnp.zeros((pad,), jnp.int32)])
    dst = jnp.concatenate([dst, jnp.full((pad,), N, jnp.int32)])

    h = _ln_relu(x, ln_gamma.reshape(1, D), ln_beta.reshape(1, D))
    zrows = jnp.zeros((ROWS_PER_SUB, D), jnp.float32)
    zdeg = jnp.zeros((N_ACC,), jnp.float32)
    parts, degs = _sc_agg(h, src, dst, zrows, zdeg)
    return _combine(parts, degs.reshape(NC, N_ACC, 1), h,
                    W_l, b_l.reshape(1, D), W_r)


# speedup vs baseline: 2.5536x; 2.5536x over previous
import functools
import inspect

import jax
import jax.numpy as jnp
from jax import lax
from jax.experimental import pallas as pl
from jax.experimental.pallas import tpu as pltpu
from jax.experimental.pallas import tpu_sc as plsc

N = 10000
D = 128
E = 320000
NC, NS = 2, 16
NW = NC * NS
CHUNK = 120
NBUF = 3
NIDX = 6
CHUNKS_PER_TILE = 84
PER_TILE = CHUNKS_PER_TILE * CHUNK
E_PAD = PER_TILE * NW
N_ACC = 10112
ROWS_PER_SUB = N_ACC // NS

_BM = 1000


def _ln_relu_body(x_ref, g_ref, b_ref, o_ref):
    x = x_ref[...]
    mu = jnp.mean(x, axis=1, keepdims=True)
    var = jnp.mean((x - mu) ** 2, axis=1, keepdims=True)
    h = (x - mu) * lax.rsqrt(var + 1e-5) * g_ref[...] + b_ref[...]
    o_ref[...] = jnp.maximum(h, 0.0)


def _ln_relu(x, g, b):
    return pl.pallas_call(
        _ln_relu_body,
        out_shape=jax.ShapeDtypeStruct((N, D), jnp.float32),
        grid=(N // _BM,),
        in_specs=[
            pl.BlockSpec((_BM, D), lambda i: (i, 0)),
            pl.BlockSpec((1, D), lambda i: (0, 0)),
            pl.BlockSpec((1, D), lambda i: (0, 0)),
        ],
        out_specs=pl.BlockSpec((_BM, D), lambda i: (i, 0)),
    )(x, g, b)


def _make_mesh():
    kw = {}
    params = inspect.signature(plsc.VectorSubcoreMesh).parameters
    if "num_cores" in params:
        kw["num_cores"] = NC
    if "num_subcores" in params:
        kw["num_subcores"] = NS
    return plsc.VectorSubcoreMesh(core_axis_name="c", subcore_axis_name="s", **kw)


def _sc_agg(h, src, dst, zrows, zdeg):
    @functools.partial(
        pl.kernel,
        out_type=[
            jax.ShapeDtypeStruct((NC, N_ACC, D), jnp.float32),
            jax.ShapeDtypeStruct((NC * N_ACC,), jnp.float32),
        ],
        mesh=_make_mesh(),
        scratch_types=(
            [pltpu.VMEM((CHUNK,), jnp.int32)] * NIDX
            + [pltpu.VMEM((CHUNK,), jnp.int32)] * NIDX
            + [pltpu.VMEM((CHUNK, D), jnp.float32)] * NBUF
            + [
                pltpu.VMEM((CHUNK,), jnp.float32),
                pltpu.VMEM_SHARED((N_ACC, D), jnp.float32),
                pltpu.VMEM_SHARED((N_ACC,), jnp.float32),
            ]
            + [pltpu.SemaphoreType.DMA] * (NIDX + 2 * NBUF)
        ),
    )
    def k(h_hbm, src_hbm, dst_hbm, zr_hbm, zd_hbm, out_hbm, deg_hbm, *refs):
        sidx = refs[:NIDX]
        didx = refs[NIDX:2 * NIDX]
        rows = refs[2 * NIDX:2 * NIDX + NBUF]
        ones = refs[2 * NIDX + NBUF]
        acc = refs[2 * NIDX + NBUF + 1]
        accd = refs[2 * NIDX + NBUF + 2]
        sembase = 2 * NIDX + NBUF + 3
        isem = refs[sembase:sembase + NIDX]
        gsem = refs[sembase + NIDX:sembase + NIDX + NBUF]
        ssem = refs[sembase + NIDX + NBUF:sembase + NIDX + 2 * NBUF]
        cid = lax.axis_index("c")
        sid = lax.axis_index("s")
        for i in range(0, CHUNK - 15, 16):
            ones[pl.ds(i, 16)] = jnp.full((16,), 1.0, jnp.float32)
        if CHUNK % 16:
            ones[pl.ds(CHUNK - 16, 16)] = jnp.full((16,), 1.0, jnp.float32)
        zoff = sid * ROWS_PER_SUB
        pltpu.sync_copy(zr_hbm, acc.at[pl.ds(zoff, ROWS_PER_SUB)])

        @pl.when(sid == 0)
        def _():
            pltpu.sync_copy(zd_hbm, accd)

        plsc.subcore_barrier()

        base = cid * (NS * PER_TILE) + sid * PER_TILE

        def idx_load(c, j, sync):
            s_slice = src_hbm.at[pl.ds(base + c * CHUNK, CHUNK)]
            d_slice = dst_hbm.at[pl.ds(base + c * CHUNK, CHUNK)]
            if sync:
                pltpu.sync_copy(s_slice, sidx[j])
                pltpu.sync_copy(d_slice, didx[j])
            else:
                pltpu.async_copy(s_slice, sidx[j], isem[j])
                pltpu.async_copy(d_slice, didx[j], isem[j])

        def idx_wait(j):
            pltpu.make_async_copy(src_hbm.at[pl.ds(0, CHUNK)], sidx[j],
                                  isem[j]).wait()
            pltpu.make_async_copy(dst_hbm.at[pl.ds(0, CHUNK)], didx[j],
                                  isem[j]).wait()

        for j in range(NBUF - 1):
            idx_load(j, j, sync=True)
        for j in range(NBUF - 1, NIDX):
            idx_load(j, j, sync=False)
        for b in range(NBUF - 1):
            pltpu.async_copy(h_hbm.at[sidx[b]], rows[b], gsem[b])

        @pl.loop(0, CHUNKS_PER_TILE, step=NIDX)
        def _(c0):
            for j in range(NIDX):
                c = c0 + j
                b = j % NBUF
                bp = (j - 1) % NBUF
                pltpu.make_async_copy(h_hbm.at[pl.ds(0, CHUNK)], rows[b],
                                      gsem[b]).wait()
                pltpu.async_copy(rows[b], acc.at[didx[j]], ssem[b], add=True)
                pltpu.sync_copy(ones, accd.at[didx[j]], add=True)

                @pl.when(c + NBUF - 1 < CHUNKS_PER_TILE)
                def _():
                    @pl.when(c > 0)
                    def _():
                        pltpu.make_async_copy(rows[bp], acc.at[didx[j]],
                                              ssem[bp]).wait()
                    idx_wait((j + NBUF - 1) % NIDX)
                    pltpu.async_copy(h_hbm.at[sidx[(j + NBUF - 1) % NIDX]],
                                     rows[bp], gsem[bp])

                    @pl.when((c > 0) & (c - 1 + NIDX < CHUNKS_PER_TILE))
                    def _():
                        idx_load(c - 1 + NIDX, (j - 1) % NIDX, sync=False)

        for t in range(NBUF):
            b = (CHUNKS_PER_TILE - 1 - t) % NBUF
            pltpu.make_async_copy(rows[b], acc.at[didx[0]], ssem[b]).wait()

        plsc.subcore_barrier()
        pltpu.sync_copy(
            acc.at[pl.ds(zoff, ROWS_PER_SUB)],
            out_hbm.at[cid, pl.ds(zoff, ROWS_PER_SUB)],
        )
        @pl.when(sid == 0)
        def _():
            pltpu.sync_copy(accd, deg_hbm.at[pl.ds(cid * N_ACC, N_ACC)])

    return k(h, src, dst, zrows, zdeg)


def _combine_body(p_ref, d_ref, h_ref, wl_ref, bl_ref, wr_ref, o_ref):
    p = p_ref[...]
    a = p[0] + p[1]
    d = d_ref[...]
    deg = jnp.maximum(d[0] + d[1], 1.0)
    mean = a / deg
    dn = (((1,), (1,)), ((), ()))
    o_ref[...] = (
        lax.dot_general(mean, wl_ref[...], dn, precision=lax.Precision.HIGHEST,
                        preferred_element_type=jnp.float32)
        + lax.dot_general(h_ref[...], wr_ref[...], dn,
                          precision=lax.Precision.HIGHEST,
                          preferred_element_type=jnp.float32)
        + bl_ref[...]
    )


def _combine(parts, degs, h, W_l, b_l, W_r):
    return pl.pallas_call(
        _combine_body,
        out_shape=jax.ShapeDtypeStruct((N, D), jnp.float32),
        grid=(N // _BM,),
        in_specs=[
            pl.BlockSpec((NC, _BM, D), lambda i: (0, i, 0)),
            pl.BlockSpec((NC, _BM, 1), lambda i: (0, i, 0)),
            pl.BlockSpec((_BM, D), lambda i: (i, 0)),
            pl.BlockSpec((D, D), lambda i: (0, 0)),
            pl.BlockSpec((1, D), lambda i: (0, 0)),
            pl.BlockSpec((D, D), lambda i: (0, 0)),
        ],
        out_specs=pl.BlockSpec((_BM, D), lambda i: (i, 0)),
    )(parts, degs, h, W_l, b_l, W_r)


def kernel(x, edge_index, ln_gamma, ln_beta, W_l, b_l, W_r):
    src = edge_index[0].astype(jnp.int32)
    dst = edge_index[1].astype(jnp.int32)
    pad = E_PAD - E
    src = jnp.concatenate([src, jnp.zeros((pad,), jnp.int32)])
    dst = jnp.concatenate([dst, jnp.full((pad,), N, jnp.int32)])

    h = _ln_relu(x, ln_gamma.reshape(1, D), ln_beta.reshape(1, D))
    zrows = jnp.zeros((ROWS_PER_SUB, D), jnp.float32)
    zdeg = jnp.zeros((N_ACC,), jnp.float32)
    parts, degs = _sc_agg(h, src, dst, zrows, zdeg)
    return _combine(parts, degs.reshape(NC, N_ACC, 1), h,
                    W_l, b_l.reshape(1, D), W_r)
